# Initial kernel scaffold; baseline (speedup 1.0000x reference)
import jax
import jax.numpy as jnp
from jax.experimental import pallas as pl


def kernel(inputs, embedding_weight):
    raise NotImplementedError("write your pallas kernel here")


# speedup vs baseline: 1.4217x; 1.4217x over previous
import functools

import jax
import jax.numpy as jnp
from jax import lax
from jax.experimental import pallas as pl
from jax.experimental.pallas import tpu as pltpu
from jax.experimental.pallas import tpu_sc as plsc

D = 256
K = 8192
N = 65536
BM = 256
NJ = 2048
NCHUNK = K // NJ
GRID = N // BM
NTOT = float(N * D)


def _argmin_body(x_ref, xn_ref, en_ref, etb_ref, enc_ref, idx_ref, hist_ref):
    i = pl.program_id(0)
    xb = (x_ref[...] * 2.0).astype(jnp.bfloat16)
    big = jnp.int32(1 << 30)
    ms, js = [], []
    for half in range(2):
        macc = None
        jacc = None
        for c in (2 * half, 2 * half + 1):
            sl = pl.ds(c * NJ, NJ)
            mm = jnp.dot(xb, etb_ref[:, sl], preferred_element_type=jnp.float32)
            d = (xn_ref[...] + en_ref[:, sl]) - mm
            jj = lax.broadcasted_iota(jnp.int32, (BM, NJ), 1) + c * NJ
            if macc is None:
                macc, jacc = d, jj
            else:
                lt = d < macc
                macc = jnp.where(lt, d, macc)
                jacc = jnp.where(lt, jj, jacc)
        m = jnp.min(macc, axis=1, keepdims=True)
        ms.append(m)
        js.append(jnp.min(jnp.where(macc == m, jacc, big), axis=1, keepdims=True))
    a1 = ms[0].astype(jnp.bfloat16).astype(jnp.float32)
    win2 = ms[1] < a1
    jsel = jnp.where(win2, js[1], js[0])
    idx_ref[...] = jsel

    @pl.when(i == 0)
    def _():
        hist_ref[...] = jnp.zeros_like(hist_ref)

    ones8 = jnp.ones((8, BM), jnp.float32)
    for c in range(NCHUNK):
        sl = pl.ds(c * NJ, NJ)
        jj = lax.broadcasted_iota(jnp.int32, (BM, NJ), 1) + c * NJ
        oh = (jj == jsel).astype(jnp.float32)
        enc_ref[:, sl] = oh
        hist_ref[:, sl] += jnp.dot(ones8, oh, preferred_element_type=jnp.float32)


def _argmin_call(flat, xn, en2, etb):
    return pl.pallas_call(
        _argmin_body,
        grid=(GRID,),
        in_specs=[
            pl.BlockSpec((BM, D), lambda i: (i, 0)),
            pl.BlockSpec((BM, 1), lambda i: (i, 0)),
            pl.BlockSpec((1, K), lambda i: (0, 0)),
            pl.BlockSpec((D, K), lambda i: (0, 0)),
        ],
        out_specs=[
            pl.BlockSpec((BM, K), lambda i: (i, 0)),
            pl.BlockSpec((BM, 1), lambda i: (i, 0)),
            pl.BlockSpec((8, K), lambda i: (0, 0)),
        ],
        out_shape=[
            jax.ShapeDtypeStruct((N, K), jnp.float32),
            jax.ShapeDtypeStruct((N, 1), jnp.int32),
            jax.ShapeDtypeStruct((8, K), jnp.float32),
        ],
        compiler_params=pltpu.CompilerParams(
            dimension_semantics=("arbitrary",),
        ),
    )(flat, xn, en2, etb)


_NW = 32
_BPW = N // _NW
_CH = 128
_NCH = _BPW // _CH


def _gather_body(table_hbm, idx_hbm, out_hbm, idx_v, buf0, buf1, semg, semw):
    wid = lax.axis_index("s") * 2 + lax.axis_index("c")
    base = wid * _BPW
    pltpu.sync_copy(idx_hbm.at[pl.ds(base, _BPW)], idx_v)
    bufs = (buf0, buf1)
    wh = [None, None]
    g = pltpu.async_copy(table_hbm.at[idx_v.at[pl.ds(0, _CH)]], buf0, semg)
    for ch in range(_NCH):
        b = ch % 2
        g.wait()
        if ch + 1 < _NCH:
            nb = (ch + 1) % 2
            if wh[nb] is not None:
                wh[nb].wait()
            g = pltpu.async_copy(
                table_hbm.at[idx_v.at[pl.ds((ch + 1) * _CH, _CH)]],
                bufs[nb], semg)
        wh[b] = pltpu.async_copy(
            bufs[b], out_hbm.at[pl.ds(base + ch * _CH, _CH)], semw)
    for h in wh:
        if h is not None:
            h.wait()


def _gather_q(emb, idx):
    mesh = plsc.VectorSubcoreMesh(core_axis_name="c", subcore_axis_name="s")
    fn = functools.partial(
        pl.kernel,
        mesh=mesh,
        out_type=jax.ShapeDtypeStruct((N, D), jnp.float32),
        scratch_types=[
            pltpu.VMEM((_BPW,), jnp.int32),
            pltpu.VMEM((_CH, D), jnp.float32),
            pltpu.VMEM((_CH, D), jnp.float32),
            pltpu.SemaphoreType.DMA,
            pltpu.SemaphoreType.DMA,
        ],
    )(_gather_body)
    return fn(emb, idx)


def _final_body(q_ref, x_ref, hist_ref, loss_ref, perp_ref, sacc_ref):
    i = pl.program_id(0)
    d2 = (q_ref[...] - x_ref[...]) ** 2

    @pl.when(i == 0)
    def _():
        sacc_ref[...] = jnp.zeros_like(sacc_ref)

    sacc_ref[...] += d2

    @pl.when(i == GRID - 1)
    def _():
        m = jnp.sum(sacc_ref[...]) * (1.0 / NTOT)
        loss_ref[...] = jnp.reshape(m + 0.25 * m, (1, 1))
        p = hist_ref[0:1, :] * (1.0 / N)
        ent = -jnp.sum(p * jnp.log(p + 1e-10))
        perp_ref[...] = jnp.reshape(jnp.exp(ent), (1, 1))


def _final_call(q, flat, hist8):
    return pl.pallas_call(
        _final_body,
        grid=(GRID,),
        in_specs=[
            pl.BlockSpec((BM, D), lambda i: (i, 0)),
            pl.BlockSpec((BM, D), lambda i: (i, 0)),
            pl.BlockSpec((8, K), lambda i: (0, 0)),
        ],
        out_specs=[
            pl.BlockSpec((1, 1), lambda i: (0, 0)),
            pl.BlockSpec((1, 1), lambda i: (0, 0)),
        ],
        out_shape=[
            jax.ShapeDtypeStruct((1, 1), jnp.float32),
            jax.ShapeDtypeStruct((1, 1), jnp.float32),
        ],
        scratch_shapes=[pltpu.VMEM((BM, D), jnp.float32)],
        compiler_params=pltpu.CompilerParams(
            dimension_semantics=("arbitrary",),
        ),
    )(q, flat, hist8)


def kernel(inputs, embedding_weight):
    input_shape = inputs.shape
    flat = inputs.reshape(-1, D)
    xn = jnp.sum(flat ** 2, axis=1, keepdims=True)
    en2 = jnp.sum(embedding_weight ** 2, axis=1).reshape(1, K)
    etb = embedding_weight.T

    encodings, idx, hist8 = _argmin_call(flat, xn, en2, etb)
    q = _gather_q(embedding_weight, idx.reshape(N))
    loss, perp = _final_call(q, flat, hist8)

    quantized_st = q.reshape(input_shape)
    return (loss.reshape(()), quantized_st, perp.reshape(()), encodings)
